# baseline (speedup 1.0000x reference)
import functools

import jax
import jax.numpy as jnp
from jax import lax
from jax.experimental import pallas as pl
from jax.experimental.pallas import tpu as pltpu
from jax.experimental.pallas import tpu_sc as plsc

B = 16384
D = 64
DP = 2 * D
NUM_CORES = 2
NUM_SUBCORES = 16
NW = NUM_CORES * NUM_SUBCORES
BW = B // NW
NCHUNK = 2
CW = BW // NCHUNK


def _build():
    mesh = plsc.VectorSubcoreMesh(core_axis_name="c", subcore_axis_name="s")

    @functools.partial(
        pl.kernel,
        mesh=mesh,
        out_type=jax.ShapeDtypeStruct((3 * B, DP), jnp.float32),
        scratch_types=[
            pltpu.VMEM((CW,), jnp.int32),
            pltpu.VMEM((CW,), jnp.int32),
            pltpu.VMEM((CW,), jnp.int32),
            pltpu.VMEM((CW, DP), jnp.float32),
            pltpu.VMEM((CW, DP), jnp.float32),
            pltpu.VMEM((CW, DP), jnp.float32),
            pltpu.SemaphoreType.DMA,
        ],
    )
    def k(hp_h, rp_h, tp_h, ent2_h, rel2_h, out_h, ih, ir, it, bh, br, bt, sem):
        wid = lax.axis_index("s") * NUM_CORES + lax.axis_index("c")
        for c in range(NCHUNK):
            base = wid * BW + c * CW
            pltpu.sync_copy(hp_h.at[pl.ds(base, CW)], ih)
            pltpu.sync_copy(rp_h.at[pl.ds(base, CW)], ir)
            pltpu.sync_copy(tp_h.at[pl.ds(base, CW)], it)
            ch = pltpu.async_copy(ent2_h.at[ih], bh, sem)
            cr = pltpu.async_copy(rel2_h.at[ir], br, sem)
            ct = pltpu.async_copy(ent2_h.at[it], bt, sem)
            ch.wait()
            pltpu.sync_copy(bh, out_h.at[pl.ds(base, CW)])
            cr.wait()
            pltpu.sync_copy(br, out_h.at[pl.ds(B + base, CW)])
            ct.wait()
            pltpu.sync_copy(bt, out_h.at[pl.ds(2 * B + base, CW)])

    return k


_gather = _build()


def kernel(head, rel, tail, ent_embeddings, rel_embeddings):
    ent2 = ent_embeddings.reshape(-1, DP)
    rel2 = rel_embeddings.reshape(-1, DP)
    staged = _gather(head // 2, rel // 2, tail // 2, ent2, rel2)
    pairs = staged.reshape(3, B, 2, D)
    parity = jnp.stack([head, rel, tail]) & 1
    sel = jnp.where(parity[:, :, None] == 1, pairs[:, :, 1, :], pairs[:, :, 0, :])
    return sel.transpose(1, 0, 2).reshape(B, 3 * D)


# speedup vs baseline: 1.1069x; 1.1069x over previous
import functools

import jax
import jax.numpy as jnp
from jax import lax
from jax.experimental import pallas as pl
from jax.experimental.pallas import tpu as pltpu
from jax.experimental.pallas import tpu_sc as plsc

B = 16384
D = 64
NUM_CORES = 2
NUM_SUBCORES = 16
NW = NUM_CORES * NUM_SUBCORES
BW = B // NW


def _build():
    mesh = plsc.VectorSubcoreMesh(core_axis_name="c", subcore_axis_name="s")

    @functools.partial(
        pl.kernel,
        mesh=mesh,
        compiler_params=pltpu.CompilerParams(use_tc_tiling_on_sc=False),
        out_type=jax.ShapeDtypeStruct((3 * B, D), jnp.float32),
        scratch_types=[
            pltpu.VMEM((BW,), jnp.int32),
            pltpu.VMEM((BW,), jnp.int32),
            pltpu.VMEM((BW,), jnp.int32),
            pltpu.VMEM((3 * BW, D), jnp.float32),
            pltpu.SemaphoreType.DMA,
        ],
    )
    def k(head_h, rel_h, tail_h, ent_h, rele_h, out_h, ih, ir, it, comb, sem):
        wid = lax.axis_index("s") * NUM_CORES + lax.axis_index("c")
        base = wid * BW
        pltpu.sync_copy(head_h.at[pl.ds(base, BW)], ih)
        pltpu.sync_copy(rel_h.at[pl.ds(base, BW)], ir)
        pltpu.sync_copy(tail_h.at[pl.ds(base, BW)], it)

        @pl.loop(0, BW // 16)
        def _(g):
            b0 = g * 16
            vh = ih[pl.ds(b0, 16)]
            vr = ir[pl.ds(b0, 16)]
            vt = it[pl.ds(b0, 16)]
            for j in range(16):
                b = b0 + j
                pltpu.async_copy(ent_h.at[vh[j]], comb.at[3 * b], sem)
                pltpu.async_copy(rele_h.at[vr[j]], comb.at[3 * b + 1], sem)
                pltpu.async_copy(ent_h.at[vt[j]], comb.at[3 * b + 2], sem)

        pltpu.make_async_copy(out_h.at[pl.ds(3 * base, 3 * BW)], comb, sem).wait()
        pltpu.sync_copy(comb, out_h.at[pl.ds(3 * base, 3 * BW)])

    return k


_gather = _build()


def kernel(head, rel, tail, ent_embeddings, rel_embeddings):
    out = _gather(head, rel, tail, ent_embeddings, rel_embeddings)
    return out.reshape(B, 3 * D)


# speedup vs baseline: 1.7441x; 1.5758x over previous
import functools

import jax
import jax.numpy as jnp
from jax import lax
from jax.experimental import pallas as pl
from jax.experimental.pallas import tpu as pltpu
from jax.experimental.pallas import tpu_sc as plsc

B = 16384
D = 64
NUM_CORES = 2
NUM_SUBCORES = 16
NW = NUM_CORES * NUM_SUBCORES
BW = B // NW
NCHUNK = 2
CW = BW // NCHUNK


def _build():
    mesh = plsc.VectorSubcoreMesh(core_axis_name="c", subcore_axis_name="s")

    @functools.partial(
        pl.kernel,
        mesh=mesh,
        out_type=jax.ShapeDtypeStruct((3 * B, D), jnp.float32),
        scratch_types=[
            pltpu.VMEM((CW,), jnp.int32),
            pltpu.VMEM((CW,), jnp.int32),
            pltpu.VMEM((CW,), jnp.int32),
            pltpu.VMEM((3 * CW, D), jnp.float32),
            pltpu.SemaphoreType.DMA,
        ],
    )
    def k(head_h, rel_h, tail_h, ent_h, rele_h, out_h, ih, ir, it, comb, sem):
        wid = lax.axis_index("s") * NUM_CORES + lax.axis_index("c")
        for c in range(NCHUNK):
            base = wid * BW + c * CW
            pltpu.sync_copy(head_h.at[pl.ds(base, CW)], ih)
            pltpu.sync_copy(rel_h.at[pl.ds(base, CW)], ir)
            pltpu.sync_copy(tail_h.at[pl.ds(base, CW)], it)

            @pl.loop(0, CW // 16)
            def _(g):
                b0 = g * 16
                vh = ih[pl.ds(b0, 16)]
                vr = ir[pl.ds(b0, 16)]
                vt = it[pl.ds(b0, 16)]
                for j in range(16):
                    b = b0 + j
                    pltpu.async_copy(
                        ent_h.at[pl.ds(vh[j], 1)], comb.at[pl.ds(3 * b, 1)], sem)
                    pltpu.async_copy(
                        rele_h.at[pl.ds(vr[j], 1)], comb.at[pl.ds(3 * b + 1, 1)], sem)
                    pltpu.async_copy(
                        ent_h.at[pl.ds(vt[j], 1)], comb.at[pl.ds(3 * b + 2, 1)], sem)

            pltpu.make_async_copy(
                out_h.at[pl.ds(3 * base, 3 * CW)], comb, sem).wait()
            pltpu.sync_copy(comb, out_h.at[pl.ds(3 * base, 3 * CW)])

    return k


_gather = _build()


def kernel(head, rel, tail, ent_embeddings, rel_embeddings):
    out = _gather(head, rel, tail, ent_embeddings, rel_embeddings)
    return out.reshape(B, 3 * D)
